# baseline (speedup 1.0000x reference)
import functools

import jax
import jax.numpy as jnp
from jax import lax
from jax.experimental import pallas as pl
from jax.experimental.pallas import tpu as pltpu
from jax.experimental.pallas import tpu_sc as plsc

N = 16384
D = 128
CHUNK = 128
NC = 2
NS = 16
NW = NC * NS
ROWS_PER_W = N // NW
CHUNKS_PER_W = ROWS_PER_W // CHUNK
NIDX_ROWS = N // CHUNK

_mesh = plsc.VectorSubcoreMesh(core_axis_name="c", subcore_axis_name="s")


@functools.partial(
    pl.kernel,
    mesh=_mesh,
    out_type=jax.ShapeDtypeStruct((N, 2 * D), jnp.float32),
    scratch_types=[
        pltpu.VMEM((CHUNKS_PER_W, CHUNK), jnp.int32),
        pltpu.VMEM((ROWS_PER_W, D), jnp.float32),
        pltpu.SemaphoreType.DMA,
        pltpu.SemaphoreType.DMA,
    ],
)
def _emb_concat(x_hbm, y_hbm, emb_hbm, out_hbm, idx_v, ebuf, gsem, xsem):
    wid = lax.axis_index("s") * NC + lax.axis_index("c")
    base = wid * ROWS_PER_W
    cx = pltpu.async_copy(
        x_hbm.at[pl.ds(base, ROWS_PER_W)],
        out_hbm.at[pl.ds(base, ROWS_PER_W), pl.ds(0, D)],
        xsem,
    )
    pltpu.sync_copy(y_hbm.at[pl.ds(wid * CHUNKS_PER_W, CHUNKS_PER_W)], idx_v)
    copies = []
    for j in range(CHUNKS_PER_W):
        copies.append(
            pltpu.async_copy(
                emb_hbm.at[idx_v.at[j]], ebuf.at[pl.ds(j * CHUNK, CHUNK)], gsem
            )
        )
    for c in copies:
        c.wait()
    pltpu.sync_copy(ebuf, out_hbm.at[pl.ds(base, ROWS_PER_W), pl.ds(D, D)])
    cx.wait()


def kernel(x, y, embedding):
    y2d = y.astype(jnp.int32).reshape(NIDX_ROWS, CHUNK)
    return _emb_concat(x, y2d, embedding)


# speedup vs baseline: 7.8750x; 7.8750x over previous
import functools

import jax
import jax.numpy as jnp
from jax import lax
from jax.experimental import pallas as pl
from jax.experimental.pallas import tpu as pltpu
from jax.experimental.pallas import tpu_sc as plsc

N = 16384
D = 128
CHUNK = 128
NC = 2
NS = 16
NW = NC * NS
ROWS_PER_W = N // NW
CHUNKS_PER_W = ROWS_PER_W // CHUNK
NIDX_ROWS = N // CHUNK

_mesh = plsc.VectorSubcoreMesh(core_axis_name="c", subcore_axis_name="s")


@functools.partial(
    pl.kernel,
    mesh=_mesh,
    out_type=jax.ShapeDtypeStruct((N, 2 * D), jnp.float32),
    scratch_types=[
        pltpu.VMEM((CHUNKS_PER_W, CHUNK), jnp.int32),
        pltpu.VMEM((2, CHUNK, 2 * D), jnp.float32),
        pltpu.SemaphoreType.DMA,
        pltpu.SemaphoreType.DMA,
        pltpu.SemaphoreType.DMA,
    ],
)
def _emb_concat(x_hbm, y_hbm, emb_hbm, out_hbm, idx_v, obuf, gsem, xsem, wsem):
    wid = lax.axis_index("s") * NC + lax.axis_index("c")
    base = wid * ROWS_PER_W
    pltpu.sync_copy(y_hbm.at[pl.ds(wid * CHUNKS_PER_W, CHUNKS_PER_W)], idx_v)
    loads_g = [None] * CHUNKS_PER_W
    loads_x = [None] * CHUNKS_PER_W
    writes = [None] * CHUNKS_PER_W
    for j in range(CHUNKS_PER_W):
        b = j % 2
        if j >= 2:
            writes[j - 2].wait()
        loads_g[j] = pltpu.async_copy(
            emb_hbm.at[idx_v.at[j]], obuf.at[b, :, pl.ds(D, D)], gsem
        )
        loads_x[j] = pltpu.async_copy(
            x_hbm.at[pl.ds(base + j * CHUNK, CHUNK)],
            obuf.at[b, :, pl.ds(0, D)],
            xsem,
        )
        if j >= 1:
            loads_g[j - 1].wait()
            loads_x[j - 1].wait()
            writes[j - 1] = pltpu.async_copy(
                obuf.at[(j - 1) % 2],
                out_hbm.at[pl.ds(base + (j - 1) * CHUNK, CHUNK)],
                wsem,
            )
    j = CHUNKS_PER_W - 1
    loads_g[j].wait()
    loads_x[j].wait()
    writes[j] = pltpu.async_copy(
        obuf.at[j % 2], out_hbm.at[pl.ds(base + j * CHUNK, CHUNK)], wsem
    )
    writes[j - 1].wait()
    writes[j].wait()


def kernel(x, y, embedding):
    y2d = y.astype(jnp.int32).reshape(NIDX_ROWS, CHUNK)
    return _emb_concat(x, y2d, embedding)
